# Initial kernel scaffold; baseline (speedup 1.0000x reference)
import jax
import jax.numpy as jnp
from jax.experimental import pallas as pl


def kernel(content, labels):
    raise NotImplementedError("write your pallas kernel here")


# speedup vs baseline: 1.6491x; 1.6491x over previous
import jax
import jax.numpy as jnp
from jax import lax
from jax.experimental import pallas as pl
from jax.experimental.pallas import tpu as pltpu

_EPS = 1e-05
_A1 = 0.3
_A2 = 0.3
_W0 = 1.0 - _A1 - _A2


def _style_src(labels):
    key = jax.random.key(42)
    b = labels.shape[0]
    src_in0 = jnp.zeros(b, dtype=jnp.int32)
    src_out0 = jnp.zeros(b, dtype=jnp.int32)

    def body(carry, label):
        key, src_in, src_out = carry
        mask = labels == label
        present = jnp.any(mask)

        def do_split(k):
            s = jax.random.split(k, 3)
            return s[0], s[1], s[2]

        def no_split(k):
            return k, k, k

        new_key, kin, kout = lax.cond(present, do_split, no_split, key)

        count = jnp.sum(mask)
        idx_in_sorted = jnp.argsort(~mask)
        idx_out_sorted = jnp.argsort(mask)

        j = jnp.where(count > 1, jax.random.randint(kin, (), 0, count - 1), 0)
        jo = jax.random.randint(kout, (), 0, (b - count) - 1)

        pick_in = idx_in_sorted[j].astype(jnp.int32)
        pick_out = idx_out_sorted[jo].astype(jnp.int32)

        src_in = jnp.where(mask, pick_in, src_in)
        src_out = jnp.where(mask, pick_out, src_out)
        return (new_key, src_in, src_out), None

    (_, src_in, src_out), _ = lax.scan(
        body, (key, src_in0, src_out0), jnp.arange(10)
    )
    return src_in, src_out


def _stats_body(x_ref, mean_ref, std_ref):
    x = x_ref[...]
    n = x.shape[-1]
    mean = jnp.mean(x, axis=-1)
    d = x - mean[:, :, None]
    var = jnp.sum(d * d, axis=-1) / (n - 1)
    mean_ref[...] = mean
    std_ref[...] = jnp.sqrt(var + _EPS)


def _apply_body(si_ref, so_ref, x_ref, mb_ref, sb_ref, mi_ref, sti_ref,
                mo_ref, sto_ref, out_ref):
    xm = mb_ref[0, 0, :]
    xs = sb_ref[0, 0, :]
    mi = mi_ref[0, 0, :]
    si = sti_ref[0, 0, :]
    mo = mo_ref[0, 0, :]
    so = sto_ref[0, 0, :]
    scale = (_W0 * xs + _A1 * si + _A2 * so) / xs
    shift = (_W0 * xm + _A1 * mi + _A2 * mo) - xm * scale
    out_ref[0] = x_ref[0] * scale[:, None] + shift[:, None]


def kernel(content, labels):
    b, c, h, w = content.shape
    hw = h * w
    x = content.reshape(b, c, hw)
    src_in, src_out = _style_src(labels)

    bb = 8
    mean, std = pl.pallas_call(
        _stats_body,
        grid=(b // bb,),
        in_specs=[pl.BlockSpec((bb, c, hw), lambda i: (i, 0, 0))],
        out_specs=[
            pl.BlockSpec((bb, c), lambda i: (i, 0)),
            pl.BlockSpec((bb, c), lambda i: (i, 0)),
        ],
        out_shape=[
            jax.ShapeDtypeStruct((b, c), jnp.float32),
            jax.ShapeDtypeStruct((b, c), jnp.float32),
        ],
    )(x)

    mean3 = mean.reshape(b, 1, c)
    std3 = std.reshape(b, 1, c)

    grid_spec = pltpu.PrefetchScalarGridSpec(
        num_scalar_prefetch=2,
        grid=(b,),
        in_specs=[
            pl.BlockSpec((1, c, hw), lambda i, si, so: (i, 0, 0)),
            pl.BlockSpec((1, 1, c), lambda i, si, so: (i, 0, 0)),
            pl.BlockSpec((1, 1, c), lambda i, si, so: (i, 0, 0)),
            pl.BlockSpec((1, 1, c), lambda i, si, so: (si[i], 0, 0)),
            pl.BlockSpec((1, 1, c), lambda i, si, so: (si[i], 0, 0)),
            pl.BlockSpec((1, 1, c), lambda i, si, so: (so[i], 0, 0)),
            pl.BlockSpec((1, 1, c), lambda i, si, so: (so[i], 0, 0)),
        ],
        out_specs=pl.BlockSpec((1, c, hw), lambda i, si, so: (i, 0, 0)),
    )
    out = pl.pallas_call(
        _apply_body,
        grid_spec=grid_spec,
        out_shape=jax.ShapeDtypeStruct((b, c, hw), jnp.float32),
    )(src_in, src_out, x, mean3, std3, mean3, std3, mean3, std3)
    return out.reshape(b, c, h, w)


# speedup vs baseline: 6.7746x; 4.1081x over previous
import jax
import jax.numpy as jnp
import numpy as np
from jax import lax
from jax.experimental import pallas as pl
from jax.experimental.pallas import tpu as pltpu

_EPS = 1e-05
_A1 = 0.3
_A2 = 0.3
_W0 = 1.0 - _A1 - _A2
_NUM_LABELS = 10


def _build_key_tables():
    key = jax.random.key(42)
    kins, kouts = [], []
    for _ in range(_NUM_LABELS):
        s = jax.random.split(key, 3)
        kins.append(np.asarray(jax.random.key_data(s[1])))
        kouts.append(np.asarray(jax.random.key_data(s[2])))
        key = s[0]
    return np.stack(kins), np.stack(kouts)


_KIN_DATA, _KOUT_DATA = _build_key_tables()


def _style_src(labels):
    b = labels.shape[0]
    lab_vals = jnp.arange(_NUM_LABELS, dtype=labels.dtype)
    masks = labels[None, :] == lab_vals[:, None]
    counts = jnp.sum(masks, axis=1)
    present = counts > 0
    nbefore = jnp.cumsum(present) - present

    kin_keys = jax.random.wrap_key_data(jnp.asarray(_KIN_DATA)[nbefore])
    kout_keys = jax.random.wrap_key_data(jnp.asarray(_KOUT_DATA)[nbefore])

    js = jax.vmap(lambda k, m: jax.random.randint(k, (), 0, m))(
        kin_keys, counts - 1)
    jos = jax.vmap(lambda k, m: jax.random.randint(k, (), 0, m))(
        kout_keys, (b - counts) - 1)
    j_used = jnp.where(counts > 1, js, 0)

    rank_in = jnp.cumsum(masks, axis=1) - 1
    rank_out = jnp.cumsum(~masks, axis=1) - 1
    pick_in = jnp.argmax(masks & (rank_in == j_used[:, None]),
                         axis=1).astype(jnp.int32)
    pick_out = jnp.argmax((~masks) & (rank_out == jos[:, None]),
                          axis=1).astype(jnp.int32)

    src_in = pick_in[labels]
    src_out = pick_out[labels]
    return src_in, src_out


def _stats_body(x_ref, mean_ref, std_ref):
    x = x_ref[...]
    n = x.shape[-1]
    mean = jnp.mean(x, axis=-1)
    d = x - mean[:, :, None]
    var = jnp.sum(d * d, axis=-1) / (n - 1)
    mean_ref[...] = mean
    std_ref[...] = jnp.sqrt(var + _EPS)


def _combine_body(si_ref, so_ref, mean_ref, std_ref, scale_ref, shift_ref,
                  mi_ref, sti_ref, mo_ref, sto_ref):
    b = mean_ref.shape[0]

    def gather_row(i, _):
        si = si_ref[i]
        so = so_ref[i]
        mi_ref[pl.ds(i, 1), :] = mean_ref[pl.ds(si, 1), :]
        sti_ref[pl.ds(i, 1), :] = std_ref[pl.ds(si, 1), :]
        mo_ref[pl.ds(i, 1), :] = mean_ref[pl.ds(so, 1), :]
        sto_ref[pl.ds(i, 1), :] = std_ref[pl.ds(so, 1), :]
        return _

    lax.fori_loop(0, b, gather_row, 0)

    xm = mean_ref[...]
    xs = std_ref[...]
    scale = (_W0 * xs + _A1 * sti_ref[...] + _A2 * sto_ref[...]) / xs
    scale_ref[...] = scale
    shift_ref[...] = (_W0 * xm + _A1 * mi_ref[...] + _A2 * mo_ref[...]) \
        - xm * scale


def _apply_body(x_ref, scale_ref, shift_ref, out_ref):
    s = scale_ref[...][:, :, None]
    t = shift_ref[...][:, :, None]
    out_ref[...] = x_ref[...] * s + t


def kernel(content, labels):
    b, c, h, w = content.shape
    hw = h * w
    x = content.reshape(b, c, hw)
    src_in, src_out = _style_src(labels)

    bb = 8
    mean, std = pl.pallas_call(
        _stats_body,
        grid=(b // bb,),
        in_specs=[pl.BlockSpec((bb, c, hw), lambda i: (i, 0, 0))],
        out_specs=[
            pl.BlockSpec((bb, c), lambda i: (i, 0)),
            pl.BlockSpec((bb, c), lambda i: (i, 0)),
        ],
        out_shape=[
            jax.ShapeDtypeStruct((b, c), jnp.float32),
            jax.ShapeDtypeStruct((b, c), jnp.float32),
        ],
    )(x)

    combine_spec = pltpu.PrefetchScalarGridSpec(
        num_scalar_prefetch=2,
        grid=(1,),
        in_specs=[
            pl.BlockSpec((b, c), lambda i, si, so: (0, 0)),
            pl.BlockSpec((b, c), lambda i, si, so: (0, 0)),
        ],
        out_specs=[
            pl.BlockSpec((b, c), lambda i, si, so: (0, 0)),
            pl.BlockSpec((b, c), lambda i, si, so: (0, 0)),
        ],
        scratch_shapes=[
            pltpu.VMEM((b, c), jnp.float32),
            pltpu.VMEM((b, c), jnp.float32),
            pltpu.VMEM((b, c), jnp.float32),
            pltpu.VMEM((b, c), jnp.float32),
        ],
    )
    scale, shift = pl.pallas_call(
        _combine_body,
        grid_spec=combine_spec,
        out_shape=[
            jax.ShapeDtypeStruct((b, c), jnp.float32),
            jax.ShapeDtypeStruct((b, c), jnp.float32),
        ],
    )(src_in, src_out, mean, std)

    out = pl.pallas_call(
        _apply_body,
        grid=(b // bb,),
        in_specs=[
            pl.BlockSpec((bb, c, hw), lambda i: (i, 0, 0)),
            pl.BlockSpec((bb, c), lambda i: (i, 0)),
            pl.BlockSpec((bb, c), lambda i: (i, 0)),
        ],
        out_specs=pl.BlockSpec((bb, c, hw), lambda i: (i, 0, 0)),
        out_shape=jax.ShapeDtypeStruct((b, c, hw), jnp.float32),
    )(x, scale, shift)
    return out.reshape(b, c, h, w)


# speedup vs baseline: 7.5296x; 1.1114x over previous
import jax
import jax.numpy as jnp
import numpy as np
from jax import lax
from jax.experimental import pallas as pl
from jax.experimental.pallas import tpu as pltpu

_EPS = 1e-05
_A1 = 0.3
_A2 = 0.3
_W0 = 1.0 - _A1 - _A2
_NUM_LABELS = 10

_KIN_DATA = np.array(
    [[64467757, 2916123636], [1705926158, 899080142],
     [1712723395, 2526649282], [2232176465, 33846082],
     [767915537, 735759787], [2252301940, 331845914],
     [2395792924, 649865367], [3515226245, 1150219387],
     [1308905690, 3242231867], [3647288517, 4265293960]], dtype=np.uint32)
_KOUT_DATA = np.array(
    [[2465931498, 255383827], [4095997477, 317277840],
     [91349104, 926951219], [2462096163, 4113027279],
     [3374067896, 3621954194], [1382268797, 2038861423],
     [3201614062, 502821546], [3650387604, 48903574],
     [272053746, 2003882608], [784671723, 584501553]], dtype=np.uint32)


def _style_src(labels):
    b = labels.shape[0]
    lab_vals = jnp.arange(_NUM_LABELS, dtype=labels.dtype)
    masks = labels[None, :] == lab_vals[:, None]
    counts = jnp.sum(masks, axis=1)
    present = counts > 0
    nbefore = jnp.cumsum(present) - present

    kin_keys = jax.random.wrap_key_data(jnp.asarray(_KIN_DATA)[nbefore])
    kout_keys = jax.random.wrap_key_data(jnp.asarray(_KOUT_DATA)[nbefore])

    js = jax.vmap(lambda k, m: jax.random.randint(k, (), 0, m))(
        kin_keys, counts - 1)
    jos = jax.vmap(lambda k, m: jax.random.randint(k, (), 0, m))(
        kout_keys, (b - counts) - 1)
    j_used = jnp.where(counts > 1, js, 0)

    rank_in = jnp.cumsum(masks, axis=1) - 1
    rank_out = jnp.cumsum(~masks, axis=1) - 1
    pick_in = jnp.argmax(masks & (rank_in == j_used[:, None]),
                         axis=1).astype(jnp.int32)
    pick_out = jnp.argmax((~masks) & (rank_out == jos[:, None]),
                          axis=1).astype(jnp.int32)

    src_in = pick_in[labels]
    src_out = pick_out[labels]
    return src_in, src_out


_CHUNK = 8
_NCHUNK = 8


def _fused_body(si_ref, so_ref, x_hbm, out_hbm, xbuf, mean_s, std_s,
                scale_s, shift_s, in_sems, out_sems):
    b = xbuf.shape[0]
    n = xbuf.shape[-1]

    def in_copy(k):
        sl = pl.ds(k * _CHUNK, _CHUNK)
        return pltpu.make_async_copy(x_hbm.at[sl], xbuf.at[sl],
                                     in_sems.at[k])

    def out_copy(k):
        sl = pl.ds(k * _CHUNK, _CHUNK)
        return pltpu.make_async_copy(xbuf.at[sl], out_hbm.at[sl],
                                     out_sems.at[k])

    for k in range(_NCHUNK):
        in_copy(k).start()

    for k in range(_NCHUNK):
        in_copy(k).wait()
        sl = pl.ds(k * _CHUNK, _CHUNK)
        x = xbuf[sl, :, :]
        mean = jnp.mean(x, axis=-1)
        d = x - mean[:, :, None]
        var = jnp.sum(d * d, axis=-1) / (n - 1)
        mean_s[sl, :] = mean
        std_s[sl, :] = jnp.sqrt(var + _EPS)

    def gather_row(r, carry):
        sic = si_ref[r]
        soc = so_ref[r]
        scale_s[pl.ds(r, 1), :] = \
            _A1 * std_s[pl.ds(sic, 1), :] + _A2 * std_s[pl.ds(soc, 1), :]
        shift_s[pl.ds(r, 1), :] = \
            _A1 * mean_s[pl.ds(sic, 1), :] + _A2 * mean_s[pl.ds(soc, 1), :]
        return carry

    lax.fori_loop(0, b, gather_row, 0)
    xm = mean_s[...]
    xs = std_s[...]
    scale = (_W0 * xs + scale_s[...]) / xs
    scale_s[...] = scale
    shift_s[...] = (_W0 * xm + shift_s[...]) - xm * scale

    for k in range(_NCHUNK):
        sl = pl.ds(k * _CHUNK, _CHUNK)
        s = scale_s[sl, :][:, :, None]
        t = shift_s[sl, :][:, :, None]
        xbuf[sl, :, :] = xbuf[sl, :, :] * s + t
        out_copy(k).start()

    for k in range(_NCHUNK):
        out_copy(k).wait()


def kernel(content, labels):
    b, c, h, w = content.shape
    hw = h * w
    x = content.reshape(b, c, hw)
    src_in, src_out = _style_src(labels)

    grid_spec = pltpu.PrefetchScalarGridSpec(
        num_scalar_prefetch=2,
        grid=(1,),
        in_specs=[pl.BlockSpec(memory_space=pltpu.MemorySpace.HBM)],
        out_specs=pl.BlockSpec(memory_space=pltpu.MemorySpace.HBM),
        scratch_shapes=[
            pltpu.VMEM((b, c, hw), jnp.float32),
            pltpu.VMEM((b, c), jnp.float32),
            pltpu.VMEM((b, c), jnp.float32),
            pltpu.VMEM((b, c), jnp.float32),
            pltpu.VMEM((b, c), jnp.float32),
            pltpu.SemaphoreType.DMA((_NCHUNK,)),
            pltpu.SemaphoreType.DMA((_NCHUNK,)),
        ],
    )
    out = pl.pallas_call(
        _fused_body,
        grid_spec=grid_spec,
        out_shape=jax.ShapeDtypeStruct((b, c, hw), jnp.float32),
        compiler_params=pltpu.CompilerParams(
            vmem_limit_bytes=62 * 1024 * 1024,
        ),
    )(src_in, src_out, x)
    return out.reshape(b, c, h, w)


# speedup vs baseline: 7.6151x; 1.0114x over previous
import jax
import jax.numpy as jnp
import numpy as np
from jax import lax
from jax.experimental import pallas as pl
from jax.experimental.pallas import tpu as pltpu

_EPS = 1e-05
_A1 = 0.3
_A2 = 0.3
_W0 = 1.0 - _A1 - _A2
_NUM_LABELS = 10

_KIN_DATA = np.array(
    [[64467757, 2916123636], [1705926158, 899080142],
     [1712723395, 2526649282], [2232176465, 33846082],
     [767915537, 735759787], [2252301940, 331845914],
     [2395792924, 649865367], [3515226245, 1150219387],
     [1308905690, 3242231867], [3647288517, 4265293960]], dtype=np.uint32)
_KOUT_DATA = np.array(
    [[2465931498, 255383827], [4095997477, 317277840],
     [91349104, 926951219], [2462096163, 4113027279],
     [3374067896, 3621954194], [1382268797, 2038861423],
     [3201614062, 502821546], [3650387604, 48903574],
     [272053746, 2003882608], [784671723, 584501553]], dtype=np.uint32)


def _style_src(labels):
    b = labels.shape[0]
    lab_vals = jnp.arange(_NUM_LABELS, dtype=labels.dtype)
    masks = labels[None, :] == lab_vals[:, None]
    counts = jnp.sum(masks, axis=1)
    present = counts > 0
    nbefore = jnp.cumsum(present) - present

    kin_keys = jax.random.wrap_key_data(jnp.asarray(_KIN_DATA)[nbefore])
    kout_keys = jax.random.wrap_key_data(jnp.asarray(_KOUT_DATA)[nbefore])

    js = jax.vmap(lambda k, m: jax.random.randint(k, (), 0, m))(
        kin_keys, counts - 1)
    jos = jax.vmap(lambda k, m: jax.random.randint(k, (), 0, m))(
        kout_keys, (b - counts) - 1)
    j_used = jnp.where(counts > 1, js, 0)

    rank_in = jnp.cumsum(masks, axis=1) - 1
    rank_out = jnp.cumsum(~masks, axis=1) - 1
    pick_in = jnp.argmax(masks & (rank_in == j_used[:, None]),
                         axis=1).astype(jnp.int32)
    pick_out = jnp.argmax((~masks) & (rank_out == jos[:, None]),
                          axis=1).astype(jnp.int32)

    src_in = pick_in[labels]
    src_out = pick_out[labels]
    return src_in, src_out


_CHUNK = 4
_NCHUNK = 16
_NSLOT = 3


def _fused_body(si_ref, so_ref, x_hbm, out_hbm, xbuf, mean_s, std_s,
                scale_s, shift_s, st0, st1, st2, in_sems, out_sems):
    b = xbuf.shape[0]
    n = xbuf.shape[-1]
    stages = [st0, st1, st2]

    def in_copy(k):
        sl = pl.ds(k * _CHUNK, _CHUNK)
        return pltpu.make_async_copy(x_hbm.at[sl], xbuf.at[sl],
                                     in_sems.at[k])

    def out_copy(k):
        sl = pl.ds(k * _CHUNK, _CHUNK)
        return pltpu.make_async_copy(stages[k % _NSLOT], out_hbm.at[sl],
                                     out_sems.at[k % _NSLOT])

    for k in range(_NCHUNK):
        in_copy(k).start()

    for k in range(_NCHUNK):
        in_copy(k).wait()
        sl = pl.ds(k * _CHUNK, _CHUNK)
        x = xbuf[sl, :, :]
        mean = jnp.mean(x, axis=-1)
        d = x - mean[:, :, None]
        var = jnp.sum(d * d, axis=-1) / (n - 1)
        mean_s[sl, :] = mean
        std_s[sl, :] = jnp.sqrt(var + _EPS)

    def gather_row(r, carry):
        sic = si_ref[r]
        soc = so_ref[r]
        scale_s[pl.ds(r, 1), :] = \
            _A1 * std_s[pl.ds(sic, 1), :] + _A2 * std_s[pl.ds(soc, 1), :]
        shift_s[pl.ds(r, 1), :] = \
            _A1 * mean_s[pl.ds(sic, 1), :] + _A2 * mean_s[pl.ds(soc, 1), :]
        return carry

    lax.fori_loop(0, b, gather_row, 0)
    xm = mean_s[...]
    xs = std_s[...]
    scale = (_W0 * xs + scale_s[...]) / xs
    scale_s[...] = scale
    shift_s[...] = (_W0 * xm + shift_s[...]) - xm * scale

    for k in range(_NCHUNK):
        if k >= _NSLOT:
            out_copy(k - _NSLOT).wait()
        sl = pl.ds(k * _CHUNK, _CHUNK)
        s = scale_s[sl, :][:, :, None]
        t = shift_s[sl, :][:, :, None]
        stages[k % _NSLOT][...] = xbuf[sl, :, :] * s + t
        out_copy(k).start()

    for k in range(_NCHUNK - _NSLOT, _NCHUNK):
        out_copy(k).wait()


def kernel(content, labels):
    b, c, h, w = content.shape
    hw = h * w
    x = content.reshape(b, c, hw)
    src_in, src_out = _style_src(labels)

    grid_spec = pltpu.PrefetchScalarGridSpec(
        num_scalar_prefetch=2,
        grid=(1,),
        in_specs=[pl.BlockSpec(memory_space=pltpu.MemorySpace.HBM)],
        out_specs=pl.BlockSpec(memory_space=pltpu.MemorySpace.HBM),
        scratch_shapes=[
            pltpu.VMEM((b, c, hw), jnp.float32),
            pltpu.VMEM((b, c), jnp.float32),
            pltpu.VMEM((b, c), jnp.float32),
            pltpu.VMEM((b, c), jnp.float32),
            pltpu.VMEM((b, c), jnp.float32),
            pltpu.VMEM((_CHUNK, c, hw), jnp.float32),
            pltpu.VMEM((_CHUNK, c, hw), jnp.float32),
            pltpu.VMEM((_CHUNK, c, hw), jnp.float32),
            pltpu.SemaphoreType.DMA((_NCHUNK,)),
            pltpu.SemaphoreType.DMA((_NSLOT,)),
        ],
    )
    out = pl.pallas_call(
        _fused_body,
        grid_spec=grid_spec,
        out_shape=jax.ShapeDtypeStruct((b, c, hw), jnp.float32),
        compiler_params=pltpu.CompilerParams(
            vmem_limit_bytes=63 * 1024 * 1024,
        ),
    )(src_in, src_out, x)
    return out.reshape(b, c, h, w)
